# Initial kernel scaffold; baseline (speedup 1.0000x reference)
import jax
import jax.numpy as jnp
from jax.experimental import pallas as pl


def kernel(x, edge_index, batch, W1, b1, W2, b2, Wfc, bfc):
    raise NotImplementedError("write your pallas kernel here")


# speedup vs baseline: 16.2177x; 16.2177x over previous
import functools

import jax
import jax.numpy as jnp
from jax import lax
from jax.experimental import pallas as pl
from jax.experimental.pallas import tpu as pltpu
from jax.experimental.pallas import tpu_sc as plsc

_N = 10000
_NP = 10240
_E = 320000
_D = 128
_DH = 64
_G = 64
_NC = 2
_NS = 16
_CH = 125
_EC = _E // _CH
_EPB = _EC // (_NC * _NS)
_RPT = _NP // _NS
_RH = _RPT // 2


@functools.cache
def _sc_mesh():
    return plsc.VectorSubcoreMesh(
        core_axis_name="c", subcore_axis_name="s",
        num_cores=_NC, num_subcores=_NS)


@functools.cache
def _sc_degree_kernel():
    return pl.kernel(
        _sc_degree_body,
        out_type=jax.ShapeDtypeStruct((_NC, _NP, 16), jnp.float32),
        mesh=_sc_mesh(),
        scratch_types=[
            pltpu.VMEM((_EPB, _CH), jnp.int32),
            pltpu.VMEM((128, 16), jnp.float32),
            pltpu.VMEM((_RPT, 16), jnp.float32),
            pltpu.VMEM_SHARED((_NP, 16), jnp.float32),
        ],
        compiler_params=pltpu.CompilerParams(use_tc_tiling_on_sc=False),
    )


def _sc_degree(dst, ones16):
    return _sc_degree_kernel()(dst, ones16)


def _sc_degree_body(dst_hbm, ones_hbm, out_hbm, idxv, onesv, bounce, acc_sh):
    ci = lax.axis_index("c")
    si = lax.axis_index("s")
    r0 = si * _RPT
    pltpu.sync_copy(ones_hbm.at[pl.ds(r0, _RPT)], bounce)
    pltpu.sync_copy(bounce, acc_sh.at[pl.ds(r0, _RPT)])
    pltpu.sync_copy(ones_hbm.at[pl.ds(0, 128)], onesv)
    wid = ci * _NS + si
    pltpu.sync_copy(dst_hbm.at[pl.ds(wid * _EPB, _EPB)], idxv)
    plsc.subcore_barrier()

    def body(j, carry):
        pltpu.sync_copy(onesv.at[pl.ds(0, _CH)], acc_sh.at[idxv.at[j]],
                        add=True)
        return carry

    lax.fori_loop(0, _EPB, body, 0)
    plsc.subcore_barrier()
    pltpu.sync_copy(acc_sh.at[pl.ds(r0, _RPT)], bounce)
    pltpu.sync_copy(bounce, out_hbm.at[ci, pl.ds(r0, _RPT)])


@functools.cache
def _sc_segsum_kernel():
    return pl.kernel(
        _sc_segsum_body,
        out_type=jax.ShapeDtypeStruct((2, _NC, _NP, _DH), jnp.float32),
        mesh=_sc_mesh(),
        scratch_types=[
            pltpu.VMEM((_EPB, _CH), jnp.int32),
            pltpu.VMEM((_EPB, _CH), jnp.int32),
            pltpu.VMEM((_CH, _DH), jnp.float32),
            pltpu.VMEM((_RH, _DH), jnp.float32),
            pltpu.VMEM_SHARED((_NP, _DH), jnp.float32),
            pltpu.SemaphoreType.DMA,
        ],
        compiler_params=pltpu.CompilerParams(use_tc_tiling_on_sc=False),
    )


def _sc_segsum(h_lo, h_hi, src, dst):
    return _sc_segsum_kernel()(h_lo, h_hi, src, dst)


def _sc_segsum_body(hlo_hbm, hhi_hbm, src_hbm, dst_hbm, out_hbm,
                    srcv, dstv, rows, bounce, acc_sh, sem):
    ci = lax.axis_index("c")
    si = lax.axis_index("s")
    wid = ci * _NS + si
    pltpu.sync_copy(src_hbm.at[pl.ds(wid * _EPB, _EPB)], srcv)
    pltpu.sync_copy(dst_hbm.at[pl.ds(wid * _EPB, _EPB)], dstv)
    for hf, h_hbm in enumerate((hlo_hbm, hhi_hbm)):
        for t in range(2):
            r0 = si * _RPT + t * _RH
            pltpu.sync_copy(h_hbm.at[pl.ds(r0, _RH)], bounce)
            pltpu.sync_copy(bounce, acc_sh.at[pl.ds(r0, _RH)])
        plsc.subcore_barrier()

        def body(j, carry):
            pltpu.async_copy(h_hbm.at[srcv.at[j]], rows, sem).wait()
            pltpu.sync_copy(rows, acc_sh.at[dstv.at[j]], add=True)
            return carry

        lax.fori_loop(0, _EPB, body, 0)
        plsc.subcore_barrier()
        for t in range(2):
            r0 = si * _RPT + t * _RH
            pltpu.sync_copy(acc_sh.at[pl.ds(r0, _RH)], bounce)
            pltpu.sync_copy(bounce, out_hbm.at[hf, ci, pl.ds(r0, _RH)])
        plsc.subcore_barrier()


def _dinv_from(degp_ref):
    deg = degp_ref[0] + degp_ref[1] - 1.0
    return lax.rsqrt(deg)[:, 0:1]


def _combine(acc_ref, hlo_ref, hhi_ref):
    lo = acc_ref[0, 0] + acc_ref[0, 1] - hlo_ref[...]
    hi = acc_ref[1, 0] + acc_ref[1, 1] - hhi_ref[...]
    return jnp.concatenate([lo, hi], axis=1)


def _tc_in_body(x_ref, w_ref, degp_ref, lo_ref, hi_ref):
    dinv = _dinv_from(degp_ref)
    h = jnp.dot(x_ref[...], w_ref[...], preferred_element_type=jnp.float32)
    hp = h * dinv
    lo_ref[...] = hp[:, :_DH]
    hi_ref[...] = hp[:, _DH:]


def _tc_mid_body(acc_ref, hlo_ref, hhi_ref, degp_ref, b_ref, w_ref,
                 lo_ref, hi_ref):
    dinv = _dinv_from(degp_ref)
    z = dinv * _combine(acc_ref, hlo_ref, hhi_ref) + b_ref[...]
    h1 = jnp.maximum(z, 0.0)
    hp = jnp.dot(h1, w_ref[...], preferred_element_type=jnp.float32) * dinv
    lo_ref[...] = hp[:, :_DH]
    hi_ref[...] = hp[:, _DH:]


def _tc_out_body(acc_ref, hlo_ref, hhi_ref, degp_ref, b_ref, batch_ref,
                 wfc_ref, bfc_ref, out_ref):
    dinv = _dinv_from(degp_ref)
    z = dinv * _combine(acc_ref, hlo_ref, hhi_ref) + b_ref[...]
    h2 = jnp.maximum(z, 0.0)
    seg = lax.broadcasted_iota(jnp.int32, (_NP, _G), 1)
    onehot = jnp.where(batch_ref[...] == seg, 1.0, 0.0)
    s = lax.dot_general(onehot, h2, (((0,), (0,)), ((), ())),
                        preferred_element_type=jnp.float32)
    cnt = lax.dot_general(onehot, jnp.ones((_NP, 1), jnp.float32),
                          (((0,), (0,)), ((), ())),
                          preferred_element_type=jnp.float32)
    pooled = s / jnp.maximum(cnt, 1.0)
    logits = jnp.dot(pooled, wfc_ref[...],
                     preferred_element_type=jnp.float32) + bfc_ref[...]
    m = jnp.max(logits, axis=1, keepdims=True)
    lse = jnp.log(jnp.sum(jnp.exp(logits - m), axis=1, keepdims=True)) + m
    out_ref[...] = logits - lse


_half_shapes = (jax.ShapeDtypeStruct((_NP, _DH), jnp.float32),
                jax.ShapeDtypeStruct((_NP, _DH), jnp.float32))


def _tc_in(x_pad, w1, degp):
    return pl.pallas_call(
        _tc_in_body, out_shape=_half_shapes)(x_pad, w1, degp)


def _tc_mid(acc, h_lo, h_hi, degp, b, w):
    return pl.pallas_call(
        _tc_mid_body, out_shape=_half_shapes)(acc, h_lo, h_hi, degp, b, w)


def _tc_out(acc, h_lo, h_hi, degp, b, batch_pad, wfc, bfc):
    return pl.pallas_call(
        _tc_out_body,
        out_shape=jax.ShapeDtypeStruct((_G, 10), jnp.float32),
    )(acc, h_lo, h_hi, degp, b, batch_pad, wfc, bfc)


def kernel(x, edge_index, batch, W1, b1, W2, b2, Wfc, bfc):
    src = edge_index[0].reshape(_EC, _CH)
    dst = edge_index[1].reshape(_EC, _CH)
    x_pad = jnp.pad(x, ((0, _NP - _N), (0, 0)))
    batch_pad = jnp.pad(batch, (0, _NP - _N),
                        constant_values=_G).reshape(_NP, 1)
    ones16 = jnp.ones((_NP, 16), jnp.float32)

    degp = _sc_degree(dst, ones16)
    h1_lo, h1_hi = _tc_in(x_pad, W1, degp)
    acc1 = _sc_segsum(h1_lo, h1_hi, src, dst)
    h2_lo, h2_hi = _tc_mid(acc1, h1_lo, h1_hi, degp, b1.reshape(1, _D), W2)
    acc2 = _sc_segsum(h2_lo, h2_hi, src, dst)
    return _tc_out(acc2, h2_lo, h2_hi, degp, b2.reshape(1, _D), batch_pad,
                   Wfc, bfc)


# speedup vs baseline: 23.5612x; 1.4528x over previous
import functools

import jax
import jax.numpy as jnp
from jax import lax
from jax.experimental import pallas as pl
from jax.experimental.pallas import tpu as pltpu
from jax.experimental.pallas import tpu_sc as plsc

_N = 10000
_NP = 10240
_E = 320000
_D = 128
_DH = 64
_G = 64
_NC = 2
_NS = 16
_CH = 125
_EC = _E // _CH
_EPB = _EC // (_NC * _NS)
_RPT = _NP // _NS
_RH = _RPT // 2


@functools.cache
def _sc_mesh():
    return plsc.VectorSubcoreMesh(
        core_axis_name="c", subcore_axis_name="s",
        num_cores=_NC, num_subcores=_NS)


@functools.cache
def _sc_degree_kernel():
    return pl.kernel(
        _sc_degree_body,
        out_type=jax.ShapeDtypeStruct((_NC, _NP, 16), jnp.float32),
        mesh=_sc_mesh(),
        scratch_types=[
            pltpu.VMEM((_EPB, _CH), jnp.int32),
            pltpu.VMEM((128, 16), jnp.float32),
            pltpu.VMEM((_RPT, 16), jnp.float32),
            pltpu.VMEM_SHARED((_NP, 16), jnp.float32),
        ],
        compiler_params=pltpu.CompilerParams(use_tc_tiling_on_sc=False),
    )


def _sc_degree(dst, ones16):
    return _sc_degree_kernel()(dst, ones16)


def _sc_degree_body(dst_hbm, ones_hbm, out_hbm, idxv, onesv, bounce, acc_sh):
    ci = lax.axis_index("c")
    si = lax.axis_index("s")
    r0 = si * _RPT
    pltpu.sync_copy(ones_hbm.at[pl.ds(r0, _RPT)], bounce)
    pltpu.sync_copy(bounce, acc_sh.at[pl.ds(r0, _RPT)])
    pltpu.sync_copy(ones_hbm.at[pl.ds(0, 128)], onesv)
    wid = ci * _NS + si
    pltpu.sync_copy(dst_hbm.at[pl.ds(wid * _EPB, _EPB)], idxv)
    plsc.subcore_barrier()

    def body(j, carry):
        pltpu.sync_copy(onesv.at[pl.ds(0, _CH)], acc_sh.at[idxv.at[j]],
                        add=True)
        return carry

    lax.fori_loop(0, _EPB, body, 0)
    plsc.subcore_barrier()
    pltpu.sync_copy(acc_sh.at[pl.ds(r0, _RPT)], bounce)
    pltpu.sync_copy(bounce, out_hbm.at[ci, pl.ds(r0, _RPT)])


@functools.cache
def _sc_segsum_kernel():
    return pl.kernel(
        _sc_segsum_body,
        out_type=jax.ShapeDtypeStruct((2, _NC, _NP, _DH), jnp.float32),
        mesh=_sc_mesh(),
        scratch_types=[
            pltpu.VMEM((_EPB, _CH), jnp.int32),
            pltpu.VMEM((_EPB, _CH), jnp.int32),
            pltpu.VMEM((_CH, _DH), jnp.float32),
            pltpu.VMEM((_CH, _DH), jnp.float32),
            pltpu.VMEM((_RH, _DH), jnp.float32),
            pltpu.VMEM_SHARED((_NP, _DH), jnp.float32),
            pltpu.SemaphoreType.DMA,
            pltpu.SemaphoreType.DMA,
        ],
        compiler_params=pltpu.CompilerParams(use_tc_tiling_on_sc=False),
    )


def _sc_segsum(h_lo, h_hi, src, dst):
    return _sc_segsum_kernel()(h_lo, h_hi, src, dst)


def _sc_segsum_body(hlo_hbm, hhi_hbm, src_hbm, dst_hbm, out_hbm,
                    srcv, dstv, rows0, rows1, bounce, acc_sh, sem0, sem1):
    ci = lax.axis_index("c")
    si = lax.axis_index("s")
    wid = ci * _NS + si
    pltpu.sync_copy(src_hbm.at[pl.ds(wid * _EPB, _EPB)], srcv)
    pltpu.sync_copy(dst_hbm.at[pl.ds(wid * _EPB, _EPB)], dstv)
    for hf, h_hbm in enumerate((hlo_hbm, hhi_hbm)):
        for t in range(2):
            r0 = si * _RPT + t * _RH
            pltpu.sync_copy(h_hbm.at[pl.ds(r0, _RH)], bounce)
            pltpu.sync_copy(bounce, acc_sh.at[pl.ds(r0, _RH)])
        plsc.subcore_barrier()

        pltpu.async_copy(h_hbm.at[srcv.at[0]], rows0, sem0)

        def body(i, carry):
            c0 = 2 * i
            c1 = c0 + 1
            pltpu.async_copy(h_hbm.at[srcv.at[c1]], rows1, sem1)
            pltpu.make_async_copy(h_hbm.at[srcv.at[c0]], rows0, sem0).wait()
            pltpu.sync_copy(rows0, acc_sh.at[dstv.at[c0]], add=True)

            @pl.when(i < _EPB // 2 - 1)
            def _():
                pltpu.async_copy(h_hbm.at[srcv.at[c0 + 2]], rows0, sem0)

            pltpu.make_async_copy(h_hbm.at[srcv.at[c1]], rows1, sem1).wait()
            pltpu.sync_copy(rows1, acc_sh.at[dstv.at[c1]], add=True)
            return carry

        lax.fori_loop(0, _EPB // 2, body, 0)
        plsc.subcore_barrier()
        for t in range(2):
            r0 = si * _RPT + t * _RH
            pltpu.sync_copy(acc_sh.at[pl.ds(r0, _RH)], bounce)
            pltpu.sync_copy(bounce, out_hbm.at[hf, ci, pl.ds(r0, _RH)])
        plsc.subcore_barrier()


def _dinv_from(degp_ref):
    deg = degp_ref[0] + degp_ref[1] - 1.0
    return lax.rsqrt(deg)[:, 0:1]


def _combine(acc_ref, hlo_ref, hhi_ref):
    lo = acc_ref[0, 0] + acc_ref[0, 1] - hlo_ref[...]
    hi = acc_ref[1, 0] + acc_ref[1, 1] - hhi_ref[...]
    return jnp.concatenate([lo, hi], axis=1)


def _tc_in_body(x_ref, w_ref, degp_ref, lo_ref, hi_ref):
    dinv = _dinv_from(degp_ref)
    h = jnp.dot(x_ref[...], w_ref[...], preferred_element_type=jnp.float32)
    hp = h * dinv
    lo_ref[...] = hp[:, :_DH]
    hi_ref[...] = hp[:, _DH:]


def _tc_mid_body(acc_ref, hlo_ref, hhi_ref, degp_ref, b_ref, w_ref,
                 lo_ref, hi_ref):
    dinv = _dinv_from(degp_ref)
    z = dinv * _combine(acc_ref, hlo_ref, hhi_ref) + b_ref[...]
    h1 = jnp.maximum(z, 0.0)
    hp = jnp.dot(h1, w_ref[...], preferred_element_type=jnp.float32) * dinv
    lo_ref[...] = hp[:, :_DH]
    hi_ref[...] = hp[:, _DH:]


def _tc_out_body(acc_ref, hlo_ref, hhi_ref, degp_ref, b_ref, batch_ref,
                 wfc_ref, bfc_ref, out_ref):
    dinv = _dinv_from(degp_ref)
    z = dinv * _combine(acc_ref, hlo_ref, hhi_ref) + b_ref[...]
    h2 = jnp.maximum(z, 0.0)
    seg = lax.broadcasted_iota(jnp.int32, (_NP, _G), 1)
    onehot = jnp.where(batch_ref[...] == seg, 1.0, 0.0)
    s = lax.dot_general(onehot, h2, (((0,), (0,)), ((), ())),
                        preferred_element_type=jnp.float32)
    cnt = lax.dot_general(onehot, jnp.ones((_NP, 1), jnp.float32),
                          (((0,), (0,)), ((), ())),
                          preferred_element_type=jnp.float32)
    pooled = s / jnp.maximum(cnt, 1.0)
    logits = jnp.dot(pooled, wfc_ref[...],
                     preferred_element_type=jnp.float32) + bfc_ref[...]
    m = jnp.max(logits, axis=1, keepdims=True)
    lse = jnp.log(jnp.sum(jnp.exp(logits - m), axis=1, keepdims=True)) + m
    out_ref[...] = logits - lse


_half_shapes = (jax.ShapeDtypeStruct((_NP, _DH), jnp.float32),
                jax.ShapeDtypeStruct((_NP, _DH), jnp.float32))


def _tc_in(x_pad, w1, degp):
    return pl.pallas_call(
        _tc_in_body, out_shape=_half_shapes)(x_pad, w1, degp)


def _tc_mid(acc, h_lo, h_hi, degp, b, w):
    return pl.pallas_call(
        _tc_mid_body, out_shape=_half_shapes)(acc, h_lo, h_hi, degp, b, w)


def _tc_out(acc, h_lo, h_hi, degp, b, batch_pad, wfc, bfc):
    return pl.pallas_call(
        _tc_out_body,
        out_shape=jax.ShapeDtypeStruct((_G, 10), jnp.float32),
    )(acc, h_lo, h_hi, degp, b, batch_pad, wfc, bfc)


def kernel(x, edge_index, batch, W1, b1, W2, b2, Wfc, bfc):
    src = edge_index[0].reshape(_EC, _CH)
    dst = edge_index[1].reshape(_EC, _CH)
    x_pad = jnp.pad(x, ((0, _NP - _N), (0, 0)))
    batch_pad = jnp.pad(batch, (0, _NP - _N),
                        constant_values=_G).reshape(_NP, 1)
    ones16 = jnp.ones((_NP, 16), jnp.float32)

    degp = _sc_degree(dst, ones16)
    h1_lo, h1_hi = _tc_in(x_pad, W1, degp)
    acc1 = _sc_segsum(h1_lo, h1_hi, src, dst)
    h2_lo, h2_hi = _tc_mid(acc1, h1_lo, h1_hi, degp, b1.reshape(1, _D), W2)
    acc2 = _sc_segsum(h2_lo, h2_hi, src, dst)
    return _tc_out(acc2, h2_lo, h2_hi, degp, b2.reshape(1, _D), batch_pad,
                   Wfc, bfc)


# speedup vs baseline: 24.6917x; 1.0480x over previous
import functools

import jax
import jax.numpy as jnp
from jax import lax
from jax.experimental import pallas as pl
from jax.experimental.pallas import tpu as pltpu
from jax.experimental.pallas import tpu_sc as plsc

_N = 10000
_NP = 10240
_E = 320000
_D = 128
_DH = 64
_G = 64
_NC = 2
_NS = 16
_CH = 125
_EC = _E // _CH
_EPB = _EC // (_NC * _NS)
_RPT = _NP // _NS
_RH = _RPT // 2
_NB = 4
_GA = 2


@functools.cache
def _sc_mesh():
    return plsc.VectorSubcoreMesh(
        core_axis_name="c", subcore_axis_name="s",
        num_cores=_NC, num_subcores=_NS)


@functools.cache
def _sc_degree_kernel():
    return pl.kernel(
        _sc_degree_body,
        out_type=jax.ShapeDtypeStruct((_NC, _NP, 16), jnp.float32),
        mesh=_sc_mesh(),
        scratch_types=[
            pltpu.VMEM((_EPB, _CH), jnp.int32),
            pltpu.VMEM((128, 16), jnp.float32),
            pltpu.VMEM((_RPT, 16), jnp.float32),
            pltpu.VMEM_SHARED((_NP, 16), jnp.float32),
            pltpu.SemaphoreType.DMA,
        ],
        compiler_params=pltpu.CompilerParams(use_tc_tiling_on_sc=False),
    )


def _sc_degree(dst, ones16):
    return _sc_degree_kernel()(dst, ones16)


def _sc_degree_body(dst_hbm, ones_hbm, out_hbm, idxv, onesv, bounce, acc_sh,
                    sem):
    ci = lax.axis_index("c")
    si = lax.axis_index("s")
    r0 = si * _RPT
    pltpu.sync_copy(ones_hbm.at[pl.ds(r0, _RPT)], bounce)
    pltpu.sync_copy(bounce, acc_sh.at[pl.ds(r0, _RPT)])
    pltpu.sync_copy(ones_hbm.at[pl.ds(0, 128)], onesv)
    wid = ci * _NS + si
    pltpu.sync_copy(dst_hbm.at[pl.ds(wid * _EPB, _EPB)], idxv)
    plsc.subcore_barrier()

    def body(j, carry):
        pltpu.async_copy(onesv.at[pl.ds(0, _CH)], acc_sh.at[idxv.at[j]],
                         sem, add=True)
        return carry

    lax.fori_loop(0, _EPB, body, 0)

    def drain(j, carry):
        pltpu.make_async_copy(onesv.at[pl.ds(0, _CH)],
                              acc_sh.at[idxv.at[0]], sem).wait()
        return carry

    lax.fori_loop(0, _EPB, drain, 0)
    plsc.subcore_barrier()
    pltpu.sync_copy(acc_sh.at[pl.ds(r0, _RPT)], bounce)
    pltpu.sync_copy(bounce, out_hbm.at[ci, pl.ds(r0, _RPT)])


@functools.cache
def _sc_segsum_kernel():
    return pl.kernel(
        _sc_segsum_body,
        out_type=jax.ShapeDtypeStruct((2, _NC, _NP, _DH), jnp.float32),
        mesh=_sc_mesh(),
        scratch_types=(
            [pltpu.VMEM((_EPB, _CH), jnp.int32),
             pltpu.VMEM((_EPB, _CH), jnp.int32)]
            + [pltpu.VMEM((_CH, _DH), jnp.float32) for _ in range(_NB)]
            + [pltpu.VMEM((_RH, _DH), jnp.float32)]
            + [pltpu.VMEM_SHARED((_NP, _DH), jnp.float32)]
            + [pltpu.SemaphoreType.DMA for _ in range(2 * _NB)]
        ),
        compiler_params=pltpu.CompilerParams(use_tc_tiling_on_sc=False),
    )


def _sc_segsum(h_lo, h_hi, src, dst):
    return _sc_segsum_kernel()(h_lo, h_hi, src, dst)


def _sc_segsum_body(hlo_hbm, hhi_hbm, src_hbm, dst_hbm, out_hbm,
                    srcv, dstv, *rest):
    rbufs = rest[:_NB]
    bounce = rest[_NB]
    acc_sh = rest[_NB + 1]
    gsems = rest[_NB + 2:2 * _NB + 2]
    ssems = rest[2 * _NB + 2:]
    ci = lax.axis_index("c")
    si = lax.axis_index("s")
    wid = ci * _NS + si
    pltpu.sync_copy(src_hbm.at[pl.ds(wid * _EPB, _EPB)], srcv)
    pltpu.sync_copy(dst_hbm.at[pl.ds(wid * _EPB, _EPB)], dstv)
    for hf, h_hbm in enumerate((hlo_hbm, hhi_hbm)):
        for t in range(2):
            r0 = si * _RPT + t * _RH
            pltpu.sync_copy(h_hbm.at[pl.ds(r0, _RH)], bounce)
            pltpu.sync_copy(bounce, acc_sh.at[pl.ds(r0, _RH)])
        plsc.subcore_barrier()

        for k in range(_GA):
            pltpu.async_copy(h_hbm.at[srcv.at[k]], rbufs[k], gsems[k])

        def body(i, carry):
            for k in range(_NB):
                c = _NB * i + k
                pltpu.make_async_copy(h_hbm.at[srcv.at[c]], rbufs[k],
                                      gsems[k]).wait()
                pltpu.async_copy(rbufs[k], acc_sh.at[dstv.at[c]], ssems[k],
                                 add=True)
                c2 = c + _GA
                b2 = (k + _GA) % _NB
                drain_ok = (c2 < _EPB) if k >= _GA else (
                    (c2 < _EPB) & (i > 0))

                @pl.when(drain_ok)
                def _():
                    pltpu.make_async_copy(rbufs[b2],
                                          acc_sh.at[dstv.at[0]],
                                          ssems[b2]).wait()

                @pl.when(c2 < _EPB)
                def _():
                    pltpu.async_copy(h_hbm.at[srcv.at[c2]], rbufs[b2],
                                     gsems[b2])
            return carry

        lax.fori_loop(0, _EPB // _NB, body, 0)
        for k in range(_NB):
            pltpu.make_async_copy(rbufs[k], acc_sh.at[dstv.at[0]],
                                  ssems[k]).wait()
        plsc.subcore_barrier()
        for t in range(2):
            r0 = si * _RPT + t * _RH
            pltpu.sync_copy(acc_sh.at[pl.ds(r0, _RH)], bounce)
            pltpu.sync_copy(bounce, out_hbm.at[hf, ci, pl.ds(r0, _RH)])
        plsc.subcore_barrier()


def _dinv_from(degp_ref):
    deg = degp_ref[0] + degp_ref[1] - 1.0
    return lax.rsqrt(deg)[:, 0:1]


def _combine(acc_ref, hlo_ref, hhi_ref):
    lo = acc_ref[0, 0] + acc_ref[0, 1] - hlo_ref[...]
    hi = acc_ref[1, 0] + acc_ref[1, 1] - hhi_ref[...]
    return jnp.concatenate([lo, hi], axis=1)


def _tc_in_body(x_ref, w_ref, degp_ref, lo_ref, hi_ref):
    dinv = _dinv_from(degp_ref)
    h = jnp.dot(x_ref[...], w_ref[...], preferred_element_type=jnp.float32)
    hp = h * dinv
    lo_ref[...] = hp[:, :_DH]
    hi_ref[...] = hp[:, _DH:]


def _tc_mid_body(acc_ref, hlo_ref, hhi_ref, degp_ref, b_ref, w_ref,
                 lo_ref, hi_ref):
    dinv = _dinv_from(degp_ref)
    z = dinv * _combine(acc_ref, hlo_ref, hhi_ref) + b_ref[...]
    h1 = jnp.maximum(z, 0.0)
    hp = jnp.dot(h1, w_ref[...], preferred_element_type=jnp.float32) * dinv
    lo_ref[...] = hp[:, :_DH]
    hi_ref[...] = hp[:, _DH:]


def _tc_out_body(acc_ref, hlo_ref, hhi_ref, degp_ref, b_ref, batch_ref,
                 wfc_ref, bfc_ref, out_ref):
    dinv = _dinv_from(degp_ref)
    z = dinv * _combine(acc_ref, hlo_ref, hhi_ref) + b_ref[...]
    h2 = jnp.maximum(z, 0.0)
    seg = lax.broadcasted_iota(jnp.int32, (_NP, _G), 1)
    onehot = jnp.where(batch_ref[...] == seg, 1.0, 0.0)
    s = lax.dot_general(onehot, h2, (((0,), (0,)), ((), ())),
                        preferred_element_type=jnp.float32)
    cnt = lax.dot_general(onehot, jnp.ones((_NP, 1), jnp.float32),
                          (((0,), (0,)), ((), ())),
                          preferred_element_type=jnp.float32)
    pooled = s / jnp.maximum(cnt, 1.0)
    logits = jnp.dot(pooled, wfc_ref[...],
                     preferred_element_type=jnp.float32) + bfc_ref[...]
    m = jnp.max(logits, axis=1, keepdims=True)
    lse = jnp.log(jnp.sum(jnp.exp(logits - m), axis=1, keepdims=True)) + m
    out_ref[...] = logits - lse


_half_shapes = (jax.ShapeDtypeStruct((_NP, _DH), jnp.float32),
                jax.ShapeDtypeStruct((_NP, _DH), jnp.float32))


def _tc_in(x_pad, w1, degp):
    return pl.pallas_call(
        _tc_in_body, out_shape=_half_shapes)(x_pad, w1, degp)


def _tc_mid(acc, h_lo, h_hi, degp, b, w):
    return pl.pallas_call(
        _tc_mid_body, out_shape=_half_shapes)(acc, h_lo, h_hi, degp, b, w)


def _tc_out(acc, h_lo, h_hi, degp, b, batch_pad, wfc, bfc):
    return pl.pallas_call(
        _tc_out_body,
        out_shape=jax.ShapeDtypeStruct((_G, 10), jnp.float32),
    )(acc, h_lo, h_hi, degp, b, batch_pad, wfc, bfc)


def kernel(x, edge_index, batch, W1, b1, W2, b2, Wfc, bfc):
    src = edge_index[0].reshape(_EC, _CH)
    dst = edge_index[1].reshape(_EC, _CH)
    x_pad = jnp.pad(x, ((0, _NP - _N), (0, 0)))
    batch_pad = jnp.pad(batch, (0, _NP - _N),
                        constant_values=_G).reshape(_NP, 1)
    ones16 = jnp.ones((_NP, 16), jnp.float32)

    degp = _sc_degree(dst, ones16)
    h1_lo, h1_hi = _tc_in(x_pad, W1, degp)
    acc1 = _sc_segsum(h1_lo, h1_hi, src, dst)
    h2_lo, h2_hi = _tc_mid(acc1, h1_lo, h1_hi, degp, b1.reshape(1, _D), W2)
    acc2 = _sc_segsum(h2_lo, h2_hi, src, dst)
    return _tc_out(acc2, h2_lo, h2_hi, degp, b2.reshape(1, _D), batch_pad,
                   Wfc, bfc)


# speedup vs baseline: 25.2871x; 1.0241x over previous
import functools

import jax
import jax.numpy as jnp
from jax import lax
from jax.experimental import pallas as pl
from jax.experimental.pallas import tpu as pltpu
from jax.experimental.pallas import tpu_sc as plsc

_N = 10000
_NP = 10240
_E = 320000
_D = 128
_DH = 64
_G = 64
_NC = 2
_NS = 16
_CH = 125
_EC = _E // _CH
_EPB = _EC // (_NC * _NS)
_RPT = _NP // _NS
_RH = _RPT // 2
_NB = 4
_GA = 2


@functools.cache
def _sc_mesh():
    return plsc.VectorSubcoreMesh(
        core_axis_name="c", subcore_axis_name="s",
        num_cores=_NC, num_subcores=_NS)


@functools.cache
def _sc_degree_kernel():
    return pl.kernel(
        _sc_degree_body,
        out_type=jax.ShapeDtypeStruct((_NC, _NP, 16), jnp.float32),
        mesh=_sc_mesh(),
        scratch_types=[
            pltpu.VMEM((_EPB, _CH), jnp.int32),
            pltpu.VMEM((128, 16), jnp.float32),
            pltpu.VMEM((_RPT, 16), jnp.float32),
            pltpu.VMEM_SHARED((_NP, 16), jnp.float32),
            pltpu.SemaphoreType.DMA,
        ],
        compiler_params=pltpu.CompilerParams(use_tc_tiling_on_sc=False),
    )


def _sc_degree(ei, ones16):
    return _sc_degree_kernel()(ei, ones16)


def _sc_degree_body(ei_hbm, ones_hbm, out_hbm, idxv, onesv, bounce, acc_sh,
                    sem):
    ci = lax.axis_index("c")
    si = lax.axis_index("s")
    r0 = si * _RPT
    pltpu.sync_copy(ones_hbm.at[pl.ds(r0, _RPT)], bounce)
    pltpu.sync_copy(bounce, acc_sh.at[pl.ds(r0, _RPT)])
    pltpu.sync_copy(ones_hbm.at[pl.ds(0, 128)], onesv)
    wid = ci * _NS + si
    pltpu.sync_copy(ei_hbm.at[1, pl.ds(wid * _EPB, _EPB)], idxv)
    plsc.subcore_barrier()

    def body(j, carry):
        pltpu.async_copy(onesv.at[pl.ds(0, _CH)], acc_sh.at[idxv.at[j]],
                         sem, add=True)
        return carry

    lax.fori_loop(0, _EPB, body, 0)

    def drain(j, carry):
        pltpu.make_async_copy(onesv.at[pl.ds(0, _CH)],
                              acc_sh.at[idxv.at[0]], sem).wait()
        return carry

    lax.fori_loop(0, _EPB, drain, 0)
    plsc.subcore_barrier()
    pltpu.sync_copy(acc_sh.at[pl.ds(r0, _RPT)], bounce)
    pltpu.sync_copy(bounce, out_hbm.at[ci, pl.ds(r0, _RPT)])


@functools.cache
def _sc_segsum_kernel():
    return pl.kernel(
        _sc_segsum_body,
        out_type=jax.ShapeDtypeStruct((2, _NC, _NP, _DH), jnp.float32),
        mesh=_sc_mesh(),
        scratch_types=(
            [pltpu.VMEM((_EPB, _CH), jnp.int32),
             pltpu.VMEM((_EPB, _CH), jnp.int32)]
            + [pltpu.VMEM((_CH, _DH), jnp.float32) for _ in range(_NB)]
            + [pltpu.VMEM((_RH, _DH), jnp.float32)]
            + [pltpu.VMEM_SHARED((_NP, _DH), jnp.float32)]
            + [pltpu.SemaphoreType.DMA for _ in range(2 * _NB)]
        ),
        compiler_params=pltpu.CompilerParams(use_tc_tiling_on_sc=False),
    )


def _sc_segsum(h_lo, h_hi, ei):
    return _sc_segsum_kernel()(h_lo, h_hi, ei)


def _sc_segsum_body(hlo_hbm, hhi_hbm, ei_hbm, out_hbm,
                    srcv, dstv, *rest):
    rbufs = rest[:_NB]
    bounce = rest[_NB]
    acc_sh = rest[_NB + 1]
    gsems = rest[_NB + 2:2 * _NB + 2]
    ssems = rest[2 * _NB + 2:]
    ci = lax.axis_index("c")
    si = lax.axis_index("s")
    wid = ci * _NS + si
    pltpu.sync_copy(ei_hbm.at[0, pl.ds(wid * _EPB, _EPB)], srcv)
    pltpu.sync_copy(ei_hbm.at[1, pl.ds(wid * _EPB, _EPB)], dstv)
    for hf, h_hbm in enumerate((hlo_hbm, hhi_hbm)):
        for t in range(2):
            r0 = si * _RPT + t * _RH
            pltpu.sync_copy(h_hbm.at[pl.ds(r0, _RH)], bounce)
            pltpu.sync_copy(bounce, acc_sh.at[pl.ds(r0, _RH)])
        plsc.subcore_barrier()

        for k in range(_GA):
            pltpu.async_copy(h_hbm.at[srcv.at[k]], rbufs[k], gsems[k])

        def body(i, carry):
            for k in range(_NB):
                c = _NB * i + k
                pltpu.make_async_copy(h_hbm.at[srcv.at[c]], rbufs[k],
                                      gsems[k]).wait()
                pltpu.async_copy(rbufs[k], acc_sh.at[dstv.at[c]], ssems[k],
                                 add=True)
                c2 = c + _GA
                b2 = (k + _GA) % _NB
                drain_ok = (c2 < _EPB) if k >= _GA else (
                    (c2 < _EPB) & (i > 0))

                @pl.when(drain_ok)
                def _():
                    pltpu.make_async_copy(rbufs[b2],
                                          acc_sh.at[dstv.at[0]],
                                          ssems[b2]).wait()

                @pl.when(c2 < _EPB)
                def _():
                    pltpu.async_copy(h_hbm.at[srcv.at[c2]], rbufs[b2],
                                     gsems[b2])
            return carry

        lax.fori_loop(0, _EPB // _NB, body, 0)
        for k in range(_NB):
            pltpu.make_async_copy(rbufs[k], acc_sh.at[dstv.at[0]],
                                  ssems[k]).wait()
        plsc.subcore_barrier()
        for t in range(2):
            r0 = si * _RPT + t * _RH
            pltpu.sync_copy(acc_sh.at[pl.ds(r0, _RH)], bounce)
            pltpu.sync_copy(bounce, out_hbm.at[hf, ci, pl.ds(r0, _RH)])
        plsc.subcore_barrier()


def _dinv_from(degp_ref):
    deg = degp_ref[0] + degp_ref[1] - 1.0
    return lax.rsqrt(deg)[:, 0:1]


def _combine(acc_ref, hlo_ref, hhi_ref):
    lo = acc_ref[0, 0] + acc_ref[0, 1] - hlo_ref[...]
    hi = acc_ref[1, 0] + acc_ref[1, 1] - hhi_ref[...]
    return jnp.concatenate([lo, hi], axis=1)


def _tc_in_body(x_ref, w_ref, degp_ref, lo_ref, hi_ref):
    dinv = _dinv_from(degp_ref)
    h = jnp.dot(x_ref[...], w_ref[...], preferred_element_type=jnp.float32)
    hp = jnp.pad(h, ((0, _NP - _N), (0, 0))) * dinv
    lo_ref[...] = hp[:, :_DH]
    hi_ref[...] = hp[:, _DH:]


def _tc_mid_body(acc_ref, hlo_ref, hhi_ref, degp_ref, b_ref, w_ref,
                 lo_ref, hi_ref):
    dinv = _dinv_from(degp_ref)
    z = dinv * _combine(acc_ref, hlo_ref, hhi_ref) + b_ref[...]
    h1 = jnp.maximum(z, 0.0)
    hp = jnp.dot(h1, w_ref[...], preferred_element_type=jnp.float32) * dinv
    lo_ref[...] = hp[:, :_DH]
    hi_ref[...] = hp[:, _DH:]


def _tc_out_body(acc_ref, hlo_ref, hhi_ref, degp_ref, b_ref, batch_ref,
                 wfc_ref, bfc_ref, out_ref):
    dinv = _dinv_from(degp_ref)
    z = dinv * _combine(acc_ref, hlo_ref, hhi_ref) + b_ref[...]
    h2 = jnp.maximum(z[:_N], 0.0)
    seg = lax.broadcasted_iota(jnp.int32, (_N, _G), 1)
    onehot = jnp.where(batch_ref[...] == seg, 1.0, 0.0)
    s = lax.dot_general(onehot, h2, (((0,), (0,)), ((), ())),
                        preferred_element_type=jnp.float32)
    cnt = lax.dot_general(onehot, jnp.ones((_N, 1), jnp.float32),
                          (((0,), (0,)), ((), ())),
                          preferred_element_type=jnp.float32)
    pooled = s / jnp.maximum(cnt, 1.0)
    logits = jnp.dot(pooled, wfc_ref[...],
                     preferred_element_type=jnp.float32) + bfc_ref[...]
    m = jnp.max(logits, axis=1, keepdims=True)
    lse = jnp.log(jnp.sum(jnp.exp(logits - m), axis=1, keepdims=True)) + m
    out_ref[...] = logits - lse


_half_shapes = (jax.ShapeDtypeStruct((_NP, _DH), jnp.float32),
                jax.ShapeDtypeStruct((_NP, _DH), jnp.float32))


def _tc_in(x_pad, w1, degp):
    return pl.pallas_call(
        _tc_in_body, out_shape=_half_shapes)(x_pad, w1, degp)


def _tc_mid(acc, h_lo, h_hi, degp, b, w):
    return pl.pallas_call(
        _tc_mid_body, out_shape=_half_shapes)(acc, h_lo, h_hi, degp, b, w)


def _tc_out(acc, h_lo, h_hi, degp, b, batch_pad, wfc, bfc):
    return pl.pallas_call(
        _tc_out_body,
        out_shape=jax.ShapeDtypeStruct((_G, 10), jnp.float32),
    )(acc, h_lo, h_hi, degp, b, batch_pad, wfc, bfc)


def kernel(x, edge_index, batch, W1, b1, W2, b2, Wfc, bfc):
    ei = edge_index.reshape(2, _EC, _CH)
    ones16 = jnp.ones((_NP, 16), jnp.float32)

    degp = _sc_degree(ei, ones16)
    h1_lo, h1_hi = _tc_in(x, W1, degp)
    acc1 = _sc_segsum(h1_lo, h1_hi, ei)
    h2_lo, h2_hi = _tc_mid(acc1, h1_lo, h1_hi, degp, b1.reshape(1, _D), W2)
    acc2 = _sc_segsum(h2_lo, h2_hi, ei)
    return _tc_out(acc2, h2_lo, h2_hi, degp, b2.reshape(1, _D),
                   batch.reshape(_N, 1), Wfc, bfc)
